# baseline (speedup 1.0000x reference)
import functools

import jax
import jax.numpy as jnp
from jax import lax
from jax.experimental import pallas as pl
from jax.experimental.pallas import tpu as pltpu
from jax.experimental.pallas import tpu_sc as plsc

N = 10000
E = 320000
D = 128
G = 64
L = 3

NC = 2
NS = 16
NW = NC * NS
C = 64
E_PAD = 327680
EPW = E_PAD // NW
NCHUNK = EPW // C
GC = 20
NGRP = NCHUNK // GC
N_PAD = 10240
RPS = N_PAD // NS
ZR = 64

R = 1024
GRID = N_PAD // R

_f32 = jnp.float32
_i32 = jnp.int32


RING = 4


def _sc_aggr_body(t_hbm, src_hbm, dst_hbm, zeros_hbm, out_hbm,
                  srcv, dstv, rows, acc,
                  zsem, gsem0, gsem1, gsem2, gsem3,
                  isem0, isem1, osem0, osem1):
    gsem = (gsem0, gsem1, gsem2, gsem3)
    isem = (isem0, isem1)
    osem = (osem0, osem1)
    c = lax.axis_index("c")
    s = lax.axis_index("s")
    w = s * NC + c

    idx_d = {}

    def load_idx(k):
        r = k % 2
        idx_d[("s", k)] = pltpu.async_copy(src_hbm.at[w, k], srcv.at[r], isem[r])
        idx_d[("d", k)] = pltpu.async_copy(dst_hbm.at[w, k], dstv.at[r], isem[r])

    load_idx(0)
    load_idx(1)

    pltpu.async_copy(zeros_hbm, rows.at[0], zsem).wait()
    zds = [pltpu.async_copy(rows.at[0], acc.at[pl.ds(s * RPS + k * ZR, ZR)],
                            zsem)
           for k in range(RPS // ZR)]
    for d in zds:
        d.wait()

    idx_d[("s", 0)].wait()
    idx_d[("d", 0)].wait()

    gd = {}

    def gather(g):
        b, r, j = g % RING, (g // GC) % 2, g % GC
        gd[g] = pltpu.async_copy(t_hbm.at[srcv.at[r, j]], rows.at[b], gsem[b])

    for g in range(RING):
        gather(g)
    plsc.subcore_barrier()

    waited = {0}
    for g in range(NCHUNK):
        b, k, r, j = g % RING, g // GC, (g // GC) % 2, g % GC
        gd[g].wait()
        pltpu.sync_copy(rows.at[b], acc.at[dstv.at[r, j]], add=True)
        if j == GC - 1 and k + 2 < NGRP:
            load_idx(k + 2)
        p = g + RING
        if p < NCHUNK:
            kp = p // GC
            if kp not in waited:
                idx_d[("s", kp)].wait()
                idx_d[("d", kp)].wait()
                waited.add(kp)
            gather(p)
    plsc.subcore_barrier()

    outd = [None, None]
    for k in range(RPS // ZR):
        b = k % 2
        r0 = s * RPS + k * ZR
        if outd[b] is not None:
            outd[b].wait()
        pltpu.sync_copy(acc.at[pl.ds(r0, ZR)], rows.at[b])
        outd[b] = pltpu.async_copy(rows.at[b], out_hbm.at[c].at[pl.ds(r0, ZR)],
                                   osem[b])
    for d in outd:
        if d is not None:
            d.wait()


@jax.jit
def _sc_aggregate(t, src_p, dst_p, zeros_zr):
    mesh = plsc.VectorSubcoreMesh(core_axis_name="c", subcore_axis_name="s")
    return pl.kernel(
        _sc_aggr_body,
        out_type=jax.ShapeDtypeStruct((NC, N_PAD, D), _f32),
        mesh=mesh,
        scratch_types=[
            pltpu.VMEM((2, GC, C), _i32),
            pltpu.VMEM((2, GC, C), _i32),
            pltpu.VMEM((RING, C, D), _f32),
            pltpu.VMEM_SHARED((N_PAD, D), _f32),
        ] + [pltpu.SemaphoreType.DMA] * 9,
    )(t, src_p, dst_p, zeros_zr)


def _tc_first_body(h_ref, wm_ref, bm_ref, wc_ref, bc_ref, t_ref, hc_ref):
    h = h_ref[...]
    t_ref[...] = jnp.maximum(
        jnp.dot(h, wm_ref[...], preferred_element_type=_f32) + bm_ref[...], 0.0)
    hc_ref[...] = jnp.dot(h, wc_ref[...], preferred_element_type=_f32) + bc_ref[...]


def _tc_mid_body(hc_ref, p0_ref, p1_ref, wm_ref, bm_ref, wc_ref, bc_ref,
                 t_ref, hcout_ref):
    h = hc_ref[...] + p0_ref[...] + p1_ref[...]
    t_ref[...] = jnp.maximum(
        jnp.dot(h, wm_ref[...], preferred_element_type=_f32) + bm_ref[...], 0.0)
    hcout_ref[...] = jnp.dot(h, wc_ref[...], preferred_element_type=_f32) + bc_ref[...]


def _tc_final_body(hc_ref, p0_ref, p1_ref, wg_ref, bg_ref, wo_ref, bo_ref,
                   batch_ref, go_ref, emb_ref):
    i = pl.program_id(0)
    h = hc_ref[...] + p0_ref[...] + p1_ref[...]
    hg = jnp.maximum(
        jnp.dot(h, wg_ref[...], preferred_element_type=_f32) + bg_ref[...], 0.0)
    rowid = i * R + lax.broadcasted_iota(_i32, (R, 1), 0)
    hg = jnp.where(rowid < N, hg, 0.0)
    b = batch_ref[0]
    mask = (lax.broadcasted_iota(_i32, (G, R), 0) == b).astype(_f32)

    @pl.when(i == 0)
    def _():
        emb_ref[...] = jnp.zeros((G, D), _f32)

    emb_ref[...] += jnp.dot(mask, hg, preferred_element_type=_f32)

    @pl.when(i == GRID - 1)
    def _():
        go_ref[...] = (
            jnp.dot(emb_ref[...], wo_ref[...], preferred_element_type=_f32)
            + bo_ref[...])


def _row_spec():
    return pl.BlockSpec((R, D), lambda i: (i, 0))


def _w_spec():
    return pl.BlockSpec((D, D), lambda i: (0, 0))


def _b_spec():
    return pl.BlockSpec((1, D), lambda i: (0, 0))


@jax.jit
def _tc_first(h, wm_t, bm, wc_t, bc):
    return pl.pallas_call(
        _tc_first_body,
        grid=(GRID,),
        in_specs=[_row_spec(), _w_spec(), _b_spec(), _w_spec(), _b_spec()],
        out_specs=[_row_spec(), _row_spec()],
        out_shape=[jax.ShapeDtypeStruct((N, D), _f32),
                   jax.ShapeDtypeStruct((N, D), _f32)],
    )(h, wm_t, bm, wc_t, bc)


@jax.jit
def _tc_mid(hc, p0, p1, wm_t, bm, wc_t, bc):
    return pl.pallas_call(
        _tc_mid_body,
        grid=(GRID,),
        in_specs=[_row_spec(), _row_spec(), _row_spec(),
                  _w_spec(), _b_spec(), _w_spec(), _b_spec()],
        out_specs=[_row_spec(), _row_spec()],
        out_shape=[jax.ShapeDtypeStruct((N, D), _f32),
                   jax.ShapeDtypeStruct((N, D), _f32)],
    )(hc, p0, p1, wm_t, bm, wc_t, bc)


@jax.jit
def _tc_final(hc, p0, p1, wg_t, bg, wo_t, bo, batch3d):
    return pl.pallas_call(
        _tc_final_body,
        grid=(GRID,),
        in_specs=[_row_spec(), _row_spec(), _row_spec(),
                  _w_spec(), _b_spec(), _w_spec(), _b_spec(),
                  pl.BlockSpec((1, 1, R), lambda i: (i, 0, 0))],
        out_specs=[pl.BlockSpec((G, D), lambda i: (0, 0)),
                   pl.BlockSpec((G, D), lambda i: (0, 0))],
        out_shape=[jax.ShapeDtypeStruct((G, D), _f32),
                   jax.ShapeDtypeStruct((G, D), _f32)],
    )(hc, p0, p1, wg_t, bg, wo_t, bo, batch3d)


def kernel(x, edge_index, batch, W_mlp, b_mlp, W_c, b_c, W_g, b_g, W_out, b_out):
    pad = jnp.arange(E_PAD - E, dtype=_i32)
    src_p = jnp.concatenate(
        [edge_index[0].astype(_i32), pad % N]
    ).reshape(NW, NGRP, GC, C)
    dst_p = jnp.concatenate(
        [edge_index[1].astype(_i32), N + pad % (N_PAD - N)]
    ).reshape(NW, NGRP, GC, C)
    zeros_zr = jnp.zeros((ZR, D), _f32)
    batch3d = jnp.concatenate(
        [batch.astype(_i32), jnp.full((N_PAD - N,), G, _i32)]).reshape(GRID, 1, R)

    hc = x
    p0 = jnp.zeros((N_PAD, D), _f32)
    p1 = p0
    first = True
    for i in range(L):
        wm_t = W_mlp[i].T
        bm = b_mlp[i].reshape(1, D)
        wc_t = W_c[i].T
        bc = b_c[i].reshape(1, D)
        if first:
            t, hc = _tc_first(hc, wm_t, bm, wc_t, bc)
            first = False
        else:
            t, hc = _tc_mid(hc, p0, p1, wm_t, bm, wc_t, bc)
        parts = _sc_aggregate(t, src_p, dst_p, zeros_zr)
        p0, p1 = parts[0], parts[1]

    global_out, embedding = _tc_final(
        hc, p0, p1, W_g.T, b_g.reshape(1, D), W_out.T, b_out.reshape(1, D),
        batch3d)
    return (global_out, embedding)


# speedup vs baseline: 1.0633x; 1.0633x over previous
import functools

import jax
import jax.numpy as jnp
from jax import lax
from jax.experimental import pallas as pl
from jax.experimental.pallas import tpu as pltpu
from jax.experimental.pallas import tpu_sc as plsc

N = 10000
E = 320000
D = 128
G = 64
L = 3

NC = 2
NS = 16
NW = NC * NS
C = 64
E_PAD = 327680
EPW = E_PAD // NW
NCHUNK = EPW // C
GC = 20
NGRP = NCHUNK // GC
N_PAD = 10240
RPS = N_PAD // NS
ZR = 64

R = 1024
GRID = N_PAD // R

_f32 = jnp.float32
_i32 = jnp.int32


RING = 4


def _sc_aggr_body(t_hbm, src_hbm, dst_hbm, zeros_hbm, out_hbm,
                  srcv, dstv, rows, acc,
                  zsem, gsem0, gsem1, gsem2, gsem3,
                  isem0, isem1, osem0, osem1):
    gsem = (gsem0, gsem1, gsem2, gsem3)
    isem = (isem0, isem1)
    osem = (osem0, osem1)
    c = lax.axis_index("c")
    s = lax.axis_index("s")
    w = s * NC + c

    idx_d = {}

    def load_idx(k):
        r = k % 2
        idx_d[("s", k)] = pltpu.async_copy(src_hbm.at[w, k], srcv.at[r], isem[r])
        idx_d[("d", k)] = pltpu.async_copy(dst_hbm.at[w, k], dstv.at[r], isem[r])

    load_idx(0)
    load_idx(1)

    pltpu.async_copy(zeros_hbm, rows.at[0], zsem).wait()
    zds = [pltpu.async_copy(rows.at[0], acc.at[pl.ds(s * RPS + k * ZR, ZR)],
                            zsem)
           for k in range(RPS // ZR)]
    for d in zds:
        d.wait()

    idx_d[("s", 0)].wait()
    idx_d[("d", 0)].wait()

    gd = {}

    def gather(g):
        b, r, j = g % RING, (g // GC) % 2, g % GC
        gd[g] = pltpu.async_copy(t_hbm.at[srcv.at[r, j]], rows.at[b], gsem[b])

    for g in range(RING):
        gather(g)
    plsc.subcore_barrier()

    waited = {0}
    for g in range(NCHUNK):
        b, k, r, j = g % RING, g // GC, (g // GC) % 2, g % GC
        gd[g].wait()
        pltpu.sync_copy(rows.at[b], acc.at[dstv.at[r, j]], add=True)
        if j == GC - 1 and k + 2 < NGRP:
            load_idx(k + 2)
        p = g + RING
        if p < NCHUNK:
            kp = p // GC
            if kp not in waited:
                idx_d[("s", kp)].wait()
                idx_d[("d", kp)].wait()
                waited.add(kp)
            gather(p)
    plsc.subcore_barrier()

    outd = [None, None]
    for k in range(RPS // ZR):
        b = k % 2
        r0 = s * RPS + k * ZR
        if outd[b] is not None:
            outd[b].wait()
        pltpu.sync_copy(acc.at[pl.ds(r0, ZR)], rows.at[b])
        outd[b] = pltpu.async_copy(rows.at[b], out_hbm.at[c].at[pl.ds(r0, ZR)],
                                   osem[b])
    for d in outd:
        if d is not None:
            d.wait()


@jax.jit
def _sc_aggregate(t, src_p, dst_p, zeros_zr):
    mesh = plsc.VectorSubcoreMesh(core_axis_name="c", subcore_axis_name="s")
    return pl.kernel(
        _sc_aggr_body,
        out_type=jax.ShapeDtypeStruct((NC, N_PAD, D), _f32),
        mesh=mesh,
        scratch_types=[
            pltpu.VMEM((2, GC, C), _i32),
            pltpu.VMEM((2, GC, C), _i32),
            pltpu.VMEM((RING, C, D), _f32),
            pltpu.VMEM_SHARED((N_PAD, D), _f32),
        ] + [pltpu.SemaphoreType.DMA] * 9,
    )(t, src_p, dst_p, zeros_zr)


def _tc_t_body(h_ref, wm_ref, bm_ref, t_ref):
    t_ref[...] = jnp.maximum(
        jnp.dot(h_ref[...], wm_ref[...], preferred_element_type=_f32)
        + bm_ref[...], 0.0)


def _tc_hc_body(h_ref, wc_ref, bc_ref, hc_ref):
    hc_ref[...] = (jnp.dot(h_ref[...], wc_ref[...], preferred_element_type=_f32)
                   + bc_ref[...])


def _tc_crit_body(hc_ref, parts_ref, wm_ref, bm_ref, t_ref, h_ref):
    h = hc_ref[...] + parts_ref[0] + parts_ref[1]
    h_ref[...] = h
    t_ref[...] = jnp.maximum(
        jnp.dot(h, wm_ref[...], preferred_element_type=_f32) + bm_ref[...], 0.0)


def _tc_final_body(hc_ref, parts_ref, wg_ref, bg_ref, wo_ref, bo_ref,
                   batch_ref, go_ref, emb_ref):
    i = pl.program_id(0)
    h = hc_ref[...] + parts_ref[0] + parts_ref[1]
    hg = jnp.maximum(
        jnp.dot(h, wg_ref[...], preferred_element_type=_f32) + bg_ref[...], 0.0)
    rowid = i * R + lax.broadcasted_iota(_i32, (R, 1), 0)
    hg = jnp.where(rowid < N, hg, 0.0)
    b = batch_ref[0]
    mask = (lax.broadcasted_iota(_i32, (G, R), 0) == b).astype(_f32)

    @pl.when(i == 0)
    def _():
        emb_ref[...] = jnp.zeros((G, D), _f32)

    emb_ref[...] += jnp.dot(mask, hg, preferred_element_type=_f32)

    @pl.when(i == GRID - 1)
    def _():
        go_ref[...] = (
            jnp.dot(emb_ref[...], wo_ref[...], preferred_element_type=_f32)
            + bo_ref[...])


def _row_spec():
    return pl.BlockSpec((R, D), lambda i: (i, 0))


def _w_spec():
    return pl.BlockSpec((D, D), lambda i: (0, 0))


def _b_spec():
    return pl.BlockSpec((1, D), lambda i: (0, 0))


def _parts_spec():
    return pl.BlockSpec((2, R, D), lambda i: (0, i, 0))


@jax.jit
def _tc_t(h, wm_t, bm):
    return pl.pallas_call(
        _tc_t_body,
        grid=(GRID,),
        in_specs=[_row_spec(), _w_spec(), _b_spec()],
        out_specs=_row_spec(),
        out_shape=jax.ShapeDtypeStruct((N, D), _f32),
    )(h, wm_t, bm)


@jax.jit
def _tc_hc(h, wc_t, bc):
    return pl.pallas_call(
        _tc_hc_body,
        grid=(GRID,),
        in_specs=[_row_spec(), _w_spec(), _b_spec()],
        out_specs=_row_spec(),
        out_shape=jax.ShapeDtypeStruct((N, D), _f32),
    )(h, wc_t, bc)


@jax.jit
def _tc_crit(hc, parts, wm_t, bm):
    return pl.pallas_call(
        _tc_crit_body,
        grid=(GRID,),
        in_specs=[_row_spec(), _parts_spec(), _w_spec(), _b_spec()],
        out_specs=[_row_spec(), _row_spec()],
        out_shape=[jax.ShapeDtypeStruct((N, D), _f32),
                   jax.ShapeDtypeStruct((N, D), _f32)],
    )(hc, parts, wm_t, bm)


@jax.jit
def _tc_final(hc, parts, wg_t, bg, wo_t, bo, batch3d):
    return pl.pallas_call(
        _tc_final_body,
        grid=(GRID,),
        in_specs=[_row_spec(), _parts_spec(),
                  _w_spec(), _b_spec(), _w_spec(), _b_spec(),
                  pl.BlockSpec((1, 1, R), lambda i: (i, 0, 0))],
        out_specs=[pl.BlockSpec((G, D), lambda i: (0, 0)),
                   pl.BlockSpec((G, D), lambda i: (0, 0))],
        out_shape=[jax.ShapeDtypeStruct((G, D), _f32),
                   jax.ShapeDtypeStruct((G, D), _f32)],
    )(hc, parts, wg_t, bg, wo_t, bo, batch3d)


def kernel(x, edge_index, batch, W_mlp, b_mlp, W_c, b_c, W_g, b_g, W_out, b_out):
    pad = jnp.arange(E_PAD - E, dtype=_i32)
    src_p = jnp.concatenate(
        [edge_index[0].astype(_i32), pad % N]
    ).reshape(NW, NGRP, GC, C)
    dst_p = jnp.concatenate(
        [edge_index[1].astype(_i32), N + pad % (N_PAD - N)]
    ).reshape(NW, NGRP, GC, C)
    zeros_zr = jnp.zeros((ZR, D), _f32)
    batch3d = jnp.concatenate(
        [batch.astype(_i32), jnp.full((N_PAD - N,), G, _i32)]).reshape(GRID, 1, R)

    h = x
    parts = None
    for i in range(L):
        wm_t = W_mlp[i].T
        bm = b_mlp[i].reshape(1, D)
        if i == 0:
            t = _tc_t(h, wm_t, bm)
        else:
            t, h = _tc_crit(hc, parts, wm_t, bm)
        parts = _sc_aggregate(t, src_p, dst_p, zeros_zr)
        hc = _tc_hc(h, W_c[i].T, b_c[i].reshape(1, D))

    global_out, embedding = _tc_final(
        hc, parts, W_g.T, b_g.reshape(1, D), W_out.T, b_out.reshape(1, D),
        batch3d)
    return (global_out, embedding)
